# Initial kernel scaffold; baseline (speedup 1.0000x reference)
import jax
import jax.numpy as jnp
from jax.experimental import pallas as pl


def kernel(context, target, negatives, W_target, W_context):
    raise NotImplementedError("write your pallas kernel here")


# speedup vs baseline: 3.3934x; 3.3934x over previous
import functools

import jax
import jax.numpy as jnp
from jax import lax
from jax.experimental import pallas as pl
from jax.experimental.pallas import tpu as pltpu
from jax.experimental.pallas import tpu_sc as plsc

VOCAB = 1000000
DIM = 64
B = 16384
WIN = 20
NNEG = 20

NC = 2
NS = 16
LANES = 16
NW = NC * NS
CPW = B // NW
S = 32
NCHUNK = CPW // S
S20 = S * WIN
IDX_ROWS = S20 // 128
NJ = DIM // LANES

_mesh = plsc.VectorSubcoreMesh(core_axis_name="c", subcore_axis_name="s")


@functools.partial(
    pl.kernel,
    out_type=[
        jax.ShapeDtypeStruct((B,), jnp.float32),
        jax.ShapeDtypeStruct((B * NNEG,), jnp.float32),
    ],
    mesh=_mesh,
    compiler_params=pltpu.CompilerParams(
        needs_layout_passes=False, use_tc_tiling_on_sc=False),
    scratch_types=[
        pltpu.VMEM((CPW,), jnp.int32),
        pltpu.VMEM((CPW * WIN // 128, 128), jnp.int32),
        pltpu.VMEM((CPW * NNEG // 128, 128), jnp.int32),
        pltpu.VMEM((S20, DIM), jnp.float32),
        pltpu.VMEM((S20, DIM), jnp.float32),
        pltpu.VMEM((S, DIM), jnp.float32),
        pltpu.VMEM((S,), jnp.float32),
        pltpu.VMEM((S20,), jnp.float32),
        pltpu.SemaphoreType.DMA,
    ],
)
def _sc_scores(ctx_hbm, tgt_hbm, neg_hbm, wt_hbm, wc_hbm,
               pos_hbm, nout_hbm,
               cidx_v, tidx_v, nidx_v, trows_v, nrows_v, crows_v,
               pos_v, nsc_v, sem):
    wid = lax.axis_index("s") * NC + lax.axis_index("c")
    wrows = CPW * WIN // 128

    pltpu.sync_copy(ctx_hbm.at[pl.ds(wid * CPW, CPW)], cidx_v)
    pltpu.sync_copy(tgt_hbm.at[pl.ds(wid * wrows, wrows)], tidx_v)
    pltpu.sync_copy(neg_hbm.at[pl.ds(wid * wrows, wrows)], nidx_v)

    def chunk_body(c, carry):
        base = wid * CPW + c * S

        copies = []
        for j in range(IDX_ROWS):
            copies.append(pltpu.async_copy(
                wt_hbm.at[tidx_v.at[c * IDX_ROWS + j]],
                trows_v.at[pl.ds(j * 128, 128)], sem))
            copies.append(pltpu.async_copy(
                wc_hbm.at[nidx_v.at[c * IDX_ROWS + j]],
                nrows_v.at[pl.ds(j * 128, 128)], sem))
        copies.append(pltpu.async_copy(
            wc_hbm.at[cidx_v.at[pl.ds(c * S, S)]], crows_v, sem))
        for cp in copies:
            cp.wait()

        for b0 in range(0, S, LANES):
            bvec = b0 + lax.iota(jnp.int32, LANES)
            bvec20 = bvec * WIN
            zero = jnp.zeros((LANES,), jnp.float32)

            def dbody(d, acc, bvec=bvec, bvec20=bvec20):
                dvec = jnp.full((LANES,), d, dtype=jnp.int32)
                gs = [plsc.load_gather(trows_v, [bvec20 + w, dvec])
                      for w in range(WIN)]
                while len(gs) > 1:
                    nxt = [gs[i] + gs[i + 1] for i in range(0, len(gs) - 1, 2)]
                    if len(gs) % 2:
                        nxt.append(gs[-1])
                    gs = nxt
                t = gs[0]
                accp = acc[0] + t * plsc.load_gather(crows_v, [bvec, dvec])
                accn = [acc[1 + n] + t * plsc.load_gather(nrows_v, [bvec20 + n, dvec])
                        for n in range(NNEG)]
                return (accp, *accn)

            res = lax.fori_loop(0, DIM, dbody, (zero,) * (1 + NNEG))
            pos_v[pl.ds(b0, LANES)] = res[0]
            for n in range(NNEG):
                nsc_v[pl.ds(n * S + b0, LANES)] = res[1 + n]

        pltpu.sync_copy(pos_v, pos_hbm.at[pl.ds(base, S)])
        pltpu.sync_copy(nsc_v, nout_hbm.at[pl.ds((wid * NCHUNK + c) * S20, S20)])
        return carry

    lax.fori_loop(0, NCHUNK, chunk_body, 0)


def _loss_body(pos_ref, neg_ref, out_ref):
    p = jnp.clip(pos_ref[...] * (1.0 / WIN), -10.0, 10.0)
    n = jnp.clip(neg_ref[...] * (1.0 / WIN), -10.0, 10.0)
    lp = jnp.sum(jnp.log1p(jnp.exp(-p)))
    ln = jnp.sum(jnp.log1p(jnp.exp(n)))
    out_ref[...] = ((lp + ln) * (1.0 / B)).reshape(1, 1)


_loss_tc = pl.pallas_call(
    _loss_body,
    out_shape=jax.ShapeDtypeStruct((1, 1), jnp.float32),
)


@jax.jit
def kernel(context, target, negatives, W_target, W_context):
    tgt2d = target.reshape(-1, 128)
    neg2d = negatives.reshape(-1, 128)
    pos_raw, neg_raw = _sc_scores(context, tgt2d, neg2d, W_target, W_context)
    out = _loss_tc(pos_raw.reshape(128, 128), neg_raw.reshape(-1, 128))
    return out[0, 0]
